# baseline (device time: 18410 ns/iter reference)
import jax
import jax.numpy as jnp
from jax import lax
from jax.experimental import pallas as pl
from jax.experimental.pallas import tpu as pltpu

N_DEV = 8
B, S_LOC, D_MODEL = 2, 128, 512
HQ, DH = 4, 64
WINDOW = 128
SCALE = 0.125


def kernel(x, Wq, K_ext, V_ext, Wo):
    def body(x_ref, wq_ref, k_ref, v_ref, wo_ref, out_ref,
             kvsend, kvl_buf, kvr_buf, send_sems, recv_sems):
        my = lax.axis_index("i")
        left = lax.rem(my + N_DEV - 1, N_DEV)
        right = lax.rem(my + 1, N_DEV)

        barrier_sem = pltpu.get_barrier_semaphore()
        for nbr in (left, right):
            pl.semaphore_signal(
                barrier_sem, inc=1,
                device_id=(nbr,), device_id_type=pl.DeviceIdType.MESH,
            )
        pl.semaphore_wait(barrier_sem, 2)

        kvsend[0] = k_ref[...].astype(jnp.bfloat16)
        kvsend[1] = v_ref[...].astype(jnp.bfloat16)
        r_to_right = pltpu.make_async_remote_copy(
            src_ref=kvsend, dst_ref=kvl_buf,
            send_sem=send_sems.at[0], recv_sem=recv_sems.at[0],
            device_id=(right,), device_id_type=pl.DeviceIdType.MESH,
        )
        r_to_left = pltpu.make_async_remote_copy(
            src_ref=kvsend, dst_ref=kvr_buf,
            send_sem=send_sems.at[1], recv_sem=recv_sems.at[1],
            device_id=(left,), device_id_type=pl.DeviceIdType.MESH,
        )
        r_to_right.start()
        r_to_left.start()

        q = lax.dot_general(
            x_ref[...].astype(jnp.bfloat16), wq_ref[...].astype(jnp.bfloat16),
            (((2,), (0,)), ((), ())),
            preferred_element_type=jnp.float32,
        ).astype(jnp.bfloat16)
        km = kvsend[0]
        vm = kvsend[1]
        sc_m = []
        for h in range(HQ):
            sc_m.append(lax.dot_general(
                q[:, :, h * DH:(h + 1) * DH], km[:, :, h, :],
                (((2,), (2,)), ((0,), (0,))),
                preferred_element_type=jnp.float32,
            ) * SCALE)

        qi = my * S_LOC + lax.broadcasted_iota(jnp.int32, (S_LOC, S_LOC), 0)
        jj = lax.broadcasted_iota(jnp.int32, (S_LOC, S_LOC), 1)
        mask_l = jnp.abs(qi - (left * S_LOC + jj)) <= WINDOW
        mask_r = jnp.abs(qi - (right * S_LOC + jj)) <= WINDOW

        r_to_right.wait_recv()
        r_to_left.wait_recv()
        kl, vl = kvl_buf[0], kvl_buf[1]
        kr, vr = kvr_buf[0], kvr_buf[1]

        ctxs = []
        for h in range(HQ):
            q_h = q[:, :, h * DH:(h + 1) * DH]
            sc_l = lax.dot_general(
                q_h, kl[:, :, h, :], (((2,), (2,)), ((0,), (0,))),
                preferred_element_type=jnp.float32) * SCALE
            sc_r = lax.dot_general(
                q_h, kr[:, :, h, :], (((2,), (2,)), ((0,), (0,))),
                preferred_element_type=jnp.float32) * SCALE
            sc_l = jnp.where(mask_l[None], sc_l, -1e9)
            sc_r = jnp.where(mask_r[None], sc_r, -1e9)
            sc = jnp.concatenate([sc_l, sc_m[h], sc_r], axis=2)
            m = jnp.max(sc, axis=-1, keepdims=True)
            w = jnp.exp(sc - m)
            w = (w / jnp.sum(w, axis=-1, keepdims=True)).astype(jnp.bfloat16)
            v_h = jnp.concatenate(
                [vl[:, :, h, :], vm[:, :, h, :], vr[:, :, h, :]], axis=1)
            ctxs.append(lax.dot_general(
                w, v_h, (((2,), (1,)), ((0,), (0,))),
                preferred_element_type=jnp.float32))

        ctx = jnp.concatenate(ctxs, axis=2).astype(jnp.bfloat16)
        out_ref[...] = lax.dot_general(
            ctx, wo_ref[...].astype(jnp.bfloat16),
            (((2,), (0,)), ((), ())),
            preferred_element_type=jnp.float32,
        )

        r_to_right.wait_send()
        r_to_left.wait_send()

    return pl.pallas_call(
        body,
        out_shape=jax.ShapeDtypeStruct((B, S_LOC, D_MODEL), jnp.float32),
        in_specs=[pl.BlockSpec(memory_space=pltpu.VMEM)] * 5,
        out_specs=pl.BlockSpec(memory_space=pltpu.VMEM),
        scratch_shapes=[
            pltpu.VMEM((2, B, S_LOC, HQ, DH), jnp.bfloat16),
            pltpu.VMEM((2, B, S_LOC, HQ, DH), jnp.bfloat16),
            pltpu.VMEM((2, B, S_LOC, HQ, DH), jnp.bfloat16),
            pltpu.SemaphoreType.DMA((2,)),
            pltpu.SemaphoreType.DMA((2,)),
        ],
        compiler_params=pltpu.CompilerParams(collective_id=0),
    )(x, Wq, K_ext, V_ext, Wo)
